# baseline (speedup 1.0000x reference)
import functools

import jax
import jax.numpy as jnp
from jax import lax
from jax.experimental import pallas as pl
from jax.experimental.pallas import tpu as pltpu
from jax.experimental.pallas import tpu_sc as plsc

BATCH = 16384
DIM = 64

NUM_CORES = 2
NUM_SUBCORES = 16
NUM_WORKERS = NUM_CORES * NUM_SUBCORES
BPW = BATCH // NUM_WORKERS
CHUNK = 128
NCHUNK = BPW // CHUNK


def _gather_cat(nids, u_emb, v_emb):
    mesh = plsc.VectorSubcoreMesh(core_axis_name="c", subcore_axis_name="s")

    @functools.partial(
        pl.kernel,
        mesh=mesh,
        out_type=jax.ShapeDtypeStruct((BATCH, 2 * DIM), jnp.float32),
        scratch_types=[
            pltpu.VMEM((NCHUNK, CHUNK), jnp.int32),
            pltpu.VMEM((BPW, DIM), jnp.float32),
            pltpu.VMEM((BPW, DIM), jnp.float32),
            pltpu.SemaphoreType.DMA,
        ],
        compiler_params=pltpu.CompilerParams(use_tc_tiling_on_sc=False),
    )
    def k(nids_hbm, u_hbm, v_hbm, out_hbm, idx_v, rows_u, rows_v, sem):
        wid = lax.axis_index("s") * NUM_CORES + lax.axis_index("c")
        base = wid * BPW
        for j in range(NCHUNK):
            pltpu.sync_copy(nids_hbm.at[pl.ds(base + j * CHUNK, CHUNK)], idx_v.at[j])
        cps = []
        for j in range(NCHUNK):
            cps.append(
                pltpu.async_copy(
                    u_hbm.at[idx_v.at[j]], rows_u.at[pl.ds(j * CHUNK, CHUNK)], sem
                )
            )
            cps.append(
                pltpu.async_copy(
                    v_hbm.at[idx_v.at[j]], rows_v.at[pl.ds(j * CHUNK, CHUNK)], sem
                )
            )
        for cp in cps:
            cp.wait()
        pltpu.sync_copy(rows_u, out_hbm.at[pl.ds(base, BPW), pl.ds(0, DIM)])
        pltpu.sync_copy(rows_v, out_hbm.at[pl.ds(base, BPW), pl.ds(DIM, DIM)])

    return k(nids, u_emb, v_emb)


def kernel(nids, is_start, directed, u_emb, v_emb):
    return _gather_cat(nids.astype(jnp.int32), u_emb, v_emb)


# speedup vs baseline: 1.4662x; 1.4662x over previous
import functools

import jax
import jax.numpy as jnp
from jax import lax
from jax.experimental import pallas as pl
from jax.experimental.pallas import tpu as pltpu
from jax.experimental.pallas import tpu_sc as plsc

BATCH = 16384
DIM = 64

NUM_CORES = 2
NUM_SUBCORES = 16
NUM_WORKERS = NUM_CORES * NUM_SUBCORES
BPW = BATCH // NUM_WORKERS
GROUP = 8
NGROUPS = BPW // GROUP
LANES = 16


def _gather_cat(nids, u_emb, v_emb):
    mesh = plsc.VectorSubcoreMesh(core_axis_name="c", subcore_axis_name="s")

    @functools.partial(
        pl.kernel,
        mesh=mesh,
        out_type=jax.ShapeDtypeStruct((BATCH, 2 * DIM), jnp.float32),
        scratch_types=[
            pltpu.VMEM((BPW,), jnp.int32),
            pltpu.VMEM((2, GROUP, 8, DIM), jnp.float32),
            pltpu.VMEM((2, GROUP, 8, DIM), jnp.float32),
            pltpu.VMEM((BPW, 2 * DIM), jnp.float32),
            pltpu.SemaphoreType.DMA,
            pltpu.SemaphoreType.DMA,
            pltpu.SemaphoreType.DMA,
        ],
    )
    def k(nids_hbm, u_hbm, v_hbm, out_hbm, idx_v, blk_u, blk_v, cat_v, sem_i, sem_a, sem_b):
        wid = lax.axis_index("s") * NUM_CORES + lax.axis_index("c")
        base = wid * BPW
        pltpu.async_copy(nids_hbm.at[pl.ds(base, BPW)], idx_v, sem_i).wait()

        sems = (sem_a, sem_b)

        def fire(g, slot):
            ivec = idx_v[pl.ds(g * GROUP, GROUP)]
            for j in range(GROUP):
                s = ivec[j]
                blk = pl.multiple_of((s >> 3) << 3, 8)
                pltpu.async_copy(u_hbm.at[pl.ds(blk, 8)], blk_u.at[slot, j], sems[slot])
                pltpu.async_copy(v_hbm.at[pl.ds(blk, 8)], blk_v.at[slot, j], sems[slot])

        def drain_extract(g, slot):
            for j in range(GROUP):
                pltpu.make_async_copy(u_hbm.at[pl.ds(0, 8)], blk_u.at[slot, j], sems[slot]).wait()
                pltpu.make_async_copy(v_hbm.at[pl.ds(0, 8)], blk_v.at[slot, j], sems[slot]).wait()
            ivec = idx_v[pl.ds(g * GROUP, GROUP)]
            for j in range(GROUP):
                r = ivec[j] & 7
                row = g * GROUP + j
                for h in range(DIM // LANES):
                    cat_v[row, pl.ds(h * LANES, LANES)] = blk_u[slot, j, r, pl.ds(h * LANES, LANES)]
                    cat_v[row, pl.ds(DIM + h * LANES, LANES)] = blk_v[slot, j, r, pl.ds(h * LANES, LANES)]

        fire(0, 0)

        def body(t, _):
            g0 = t * 2
            fire(g0 + 1, 1)
            drain_extract(g0, 0)

            @pl.when(g0 + 2 < NGROUPS)
            def _():
                fire(g0 + 2, 0)

            drain_extract(g0 + 1, 1)
            return ()

        lax.fori_loop(0, NGROUPS // 2, body, ())
        pltpu.sync_copy(cat_v, out_hbm.at[pl.ds(base, BPW)])

    return k(nids, u_emb, v_emb)


def kernel(nids, is_start, directed, u_emb, v_emb):
    return _gather_cat(nids.astype(jnp.int32), u_emb, v_emb)
